# baseline (speedup 1.0000x reference)
import functools

import jax
import jax.numpy as jnp
from jax import lax
from jax.experimental import pallas as pl
from jax.experimental.pallas import tpu as pltpu
from jax.experimental.pallas import tpu_sc as plsc

_NUM_CLASSES = 20
_NCH = 6 + _NUM_CLASSES
_SCALE = 16.0
_IGNORE_THRESH = 0.5
_BAD_CONF_WEIGHT = 1.25
_ANCHORS = [(25.0, 50.0), (50.0, 100.0), (100.0, 200.0), (200.0, 120.0),
            (320.0, 320.0)]

_B, _NA, _NH, _NW, _MAXT = 16, 5, 32, 32, 50
_CELLS = _NH * _NW
_GRID = _NA * _CELLS
_NFIELD = 16
_TPAD = 128
_MG = _NA * 2 * _CELLS
_NROWCH = 32


def _records_body(tgt_ref, ts_ref, out_ref):
    tgt = tgt_ref[...]
    ts = ts_ref[...]
    inv_s = 1.0 / _SCALE
    gx = tgt[:, :, 0] * inv_s
    gy = tgt[:, :, 1] * inv_s
    gh = tgt[:, :, 3] * inv_s
    gw = tgt[:, :, 4] * inv_s

    tt = lax.broadcasted_iota(jnp.int32, (_B, _MAXT), 1)
    bb = lax.broadcasted_iota(jnp.int32, (_B, _MAXT), 0)
    valid = (tt < ts) & (gw != 0.0) & (gh != 0.0)

    gi = jnp.clip(gx.astype(jnp.int32), 0, _NW - 1)
    gj = jnp.clip(gy.astype(jnp.int32), 0, _NH - 1)

    a1 = (gw + 1.0) * (gh + 1.0)
    ious = []
    for aw, ah in _ANCHORS:
        aw, ah = aw / _SCALE, ah / _SCALE
        inter = (jnp.clip(jnp.minimum(gw, aw) + 1.0, 0.0, None) *
                 jnp.clip(jnp.minimum(gh, ah) + 1.0, 0.0, None))
        a2 = (aw + 1.0) * (ah + 1.0)
        ious.append(inter / (a1 + a2 - inter + 1e-16))

    best_iou = ious[0]
    best_n = jnp.zeros((_B, _MAXT), jnp.int32)
    for a in range(1, _NA):
        upd = ious[a] > best_iou
        best_n = jnp.where(upd, a, best_n)
        best_iou = jnp.where(upd, ious[a], best_iou)

    validf = valid.astype(jnp.float32)
    ign = [((iou_a > _IGNORE_THRESH) & valid).astype(jnp.float32)
           for iou_a in ious]

    aw_best = jnp.full((_B, _MAXT), _ANCHORS[0][0] / _SCALE)
    ah_best = jnp.full((_B, _MAXT), _ANCHORS[0][1] / _SCALE)
    for a in range(1, _NA):
        sel = best_n == a
        aw_best = jnp.where(sel, _ANCHORS[a][0] / _SCALE, aw_best)
        ah_best = jnp.where(sel, _ANCHORS[a][1] / _SCALE, ah_best)

    def inv_tanh(y):
        yc = jnp.clip(y, -0.999999, 0.999999)
        inner = 0.5 * jnp.log((1.0 + yc) / (1.0 - yc))
        return jnp.where(y <= -1.0, -2.0, jnp.where(y >= 1.0, 2.0, inner))

    txv = inv_tanh(gx - (gi.astype(jnp.float32) + 0.5))
    tyv = inv_tanh(gy - (gj.astype(jnp.float32) + 0.5))
    twv = jnp.log(gw / aw_best + 1e-16)
    thv = jnp.log(gh / ah_best + 1e-16)

    cidx = lax.broadcasted_iota(
        jnp.int32, (_B, _MAXT, _NUM_CLASSES), 2).astype(jnp.float32)
    label = jnp.sum(tgt[:, :, 13:13 + _NUM_CLASSES] * cidx, axis=2)

    cell = gj * _NW + gi
    key_mask = (best_n * (2 * _CELLS) + cell).astype(jnp.float32)
    key_win = (best_n * _CELLS + cell).astype(jnp.float32)
    rowid = ((bb * _NA + best_n) * _CELLS + cell).astype(jnp.float32)

    zeros = jnp.zeros((_B, _MAXT), jnp.float32)
    fields = [validf, txv, tyv, twv, thv, label,
              ign[0], ign[1], ign[2], ign[3], ign[4],
              key_mask, cell.astype(jnp.float32), rowid, key_win, zeros]
    pad = jnp.zeros((_B, _TPAD - _MAXT), jnp.float32)
    for k, f in enumerate(fields):
        out_ref[k] = jnp.concatenate([f, pad], axis=1)


def _make_records(target, target_sizes):
    return pl.pallas_call(
        _records_body,
        out_shape=jax.ShapeDtypeStruct((_NFIELD, _B, _TPAD), jnp.float32),
    )(target, target_sizes.astype(jnp.int32).reshape(_B, 1))


def _sc_body(rec_hbm, pred1_hbm, mg_out, conf_out, rows_out,
             rec_v, mg_v, win_v, cidx_v, conf_v, ridx_v, ridx2_v, rows_v,
             rowst_v, sem_c, sem_r):
    cid = lax.axis_index("c")
    sid = lax.axis_index("s")

    @pl.when(cid == 0)
    def _():
        b = sid
        lane = lax.iota(jnp.int32, 16)

        cbase = (b * _GRID) * _NCH

        def cidx_body(i, carry):
            row = cbase + i * (128 * _NCH)
            for u in range(8):
                cidx_v[i, pl.ds(u * 16, 16)] = row + (u * 16 + lane) * _NCH
            return carry

        lax.fori_loop(0, 40, cidx_body, 0)

        def cfire_body(i, carry):
            pltpu.async_copy(pred1_hbm.at[cidx_v.at[i]], conf_v.at[i], sem_c)
            return carry

        lax.fori_loop(0, 40, cfire_body, 0)

        pltpu.sync_copy(rec_hbm.at[:, b], rec_v)

        def zero_mg(i, carry):
            zero = jnp.zeros((16,), jnp.float32)
            for u in range(8):
                mg_v[pl.ds(i * 128 + u * 16, 16)] = zero
            return carry

        lax.fori_loop(0, _MG // 128, zero_mg, 0)

        def zero_win(i, carry):
            zero = jnp.zeros((16,), jnp.float32)
            for u in range(8):
                win_v[pl.ds(i * 128 + u * 16, 16)] = zero
            return carry

        lax.fori_loop(0, _GRID // 128, zero_win, 0)

        def zero_rowst(i, carry):
            zero = jnp.zeros((16,), jnp.float32)
            for u in range(8):
                rowst_v[pl.ds(i * 128 + u * 16, 16)] = zero
            return carry

        lax.fori_loop(0, _NROWCH * _TPAD // 128, zero_rowst, 0)

        def ridx_body(k, carry):
            cols = jnp.minimum(k * 16 + lane, _MAXT - 1)
            vals = plsc.load_gather(rec_v, [jnp.full((16,), 13, jnp.int32),
                                            cols])
            ridx_v[pl.ds(k * 16, 16)] = vals.astype(jnp.int32)
            return carry

        lax.fori_loop(0, 4, ridx_body, 0)

        def ridx2_body(i, carry):
            for u in range(8):
                s = i * 128 + u * 16 + lane
                t = jax.lax.shift_right_logical(s, 5)
                c = jnp.minimum(s & 31, _NCH - 1)
                rid = plsc.load_gather(ridx_v, [t])
                ridx2_v[i, pl.ds(u * 16, 16)] = rid * _NCH + c
            return carry

        lax.fori_loop(0, 13, ridx2_body, 0)

        def rfire_body(i, carry):
            pltpu.async_copy(pred1_hbm.at[ridx2_v.at[i]],
                             rows_v.at[pl.ds(i * 128, 128)], sem_r)
            return carry

        lax.fori_loop(0, 13, rfire_body, 0)

        def tgt_body(t, carry):
            v = plsc.load_gather(rec_v, [lane, jnp.full((16,), t, jnp.int32)])
            valid = v[0]
            kmask = v[11].astype(jnp.int32)
            cell = v[12].astype(jnp.int32)
            kwin = v[14].astype(jnp.int32)
            validv = jnp.full((16,), valid) > 0.0
            is_ign = (lane >= 6) & (lane < 11)
            m1 = ((lane == 0) & validv) | (is_ign & (v > 0.0))
            idx1 = jnp.where(
                lane == 0, kmask,
                jnp.where(is_ign, (lane - 6) * (2 * _CELLS) + _CELLS + cell,
                          0))
            plsc.store_scatter(mg_v, [jnp.where(m1, idx1, 0)], v, mask=m1)
            m2 = (lane == 0) & validv
            plsc.store_scatter(win_v, [jnp.where(m2, kwin, 0)],
                               jnp.full((16,), t, jnp.int32).astype(
                                   jnp.float32), mask=m2)
            return carry

        lax.fori_loop(0, _MAXT, tgt_body, 0)

        def rdrain_body(i, carry):
            pltpu.make_async_copy(pred1_hbm.at[ridx2_v.at[i]],
                                  rows_v.at[pl.ds(i * 128, 128)], sem_r).wait()
            return carry

        lax.fori_loop(0, 13, rdrain_body, 0)

        def out_body(t, carry):
            v = plsc.load_gather(rec_v, [lane, jnp.full((16,), t, jnp.int32)])
            valid = v[0]
            kwin = v[14].astype(jnp.int32)
            wv = plsc.load_gather(win_v, [jnp.full((16,), kwin, jnp.int32)])
            t_f = t.astype(jnp.float32)
            flag = jnp.where((wv[0] == t_f) & (valid > 0.0), 1.0, 0.0)
            ch1 = plsc.load_gather(rows_v, [t * 32 + lane])
            ch2 = plsc.load_gather(
                rows_v, [t * 32 + jnp.minimum(16 + lane, _NCH - 1)])
            plsc.store_scatter(rowst_v, [lane * _TPAD + t], ch1)
            plsc.store_scatter(rowst_v, [(16 + lane) * _TPAD + t], ch2,
                               mask=lane < (_NCH - 16))
            plsc.store_scatter(rowst_v, [jnp.full((16,), 26 * _TPAD, jnp.int32)
                                         + t],
                               jnp.full((16,), flag), mask=lane == 0)
            return carry

        lax.fori_loop(0, _MAXT, out_body, 0)

        def cdrain_body(i, carry):
            pltpu.make_async_copy(
                pred1_hbm.at[cidx_v.at[i]], conf_v.at[i], sem_c).wait()
            return carry

        lax.fori_loop(0, 40, cdrain_body, 0)

        pltpu.sync_copy(mg_v, mg_out.at[pl.ds(b * _MG, _MG)])
        pltpu.sync_copy(conf_v, conf_out.at[pl.ds(b * 40, 40)])
        pltpu.sync_copy(rowst_v,
                        rows_out.at[pl.ds(b * _NROWCH * _TPAD,
                                          _NROWCH * _TPAD)])


def _sc_scatter(rec, pred1):
    mesh = plsc.VectorSubcoreMesh(core_axis_name="c", subcore_axis_name="s",
                                  num_cores=2, num_subcores=16)
    fn = functools.partial(
        pl.kernel,
        out_type=(
            jax.ShapeDtypeStruct((_B * _MG,), jnp.float32),
            jax.ShapeDtypeStruct((_B * 40, 128), jnp.float32),
            jax.ShapeDtypeStruct((_B * _NROWCH * _TPAD,), jnp.float32),
        ),
        mesh=mesh,
        scratch_types=[
            pltpu.VMEM((_NFIELD, _TPAD), jnp.float32),
            pltpu.VMEM((_MG,), jnp.float32),
            pltpu.VMEM((_GRID,), jnp.float32),
            pltpu.VMEM((40, 128), jnp.int32),
            pltpu.VMEM((40, 128), jnp.float32),
            pltpu.VMEM((64,), jnp.int32),
            pltpu.VMEM((13, 128), jnp.int32),
            pltpu.VMEM((13 * 128,), jnp.float32),
            pltpu.VMEM((_NROWCH * _TPAD,), jnp.float32),
            pltpu.SemaphoreType.DMA,
            pltpu.SemaphoreType.DMA,
        ],
        compiler_params=pltpu.CompilerParams(needs_layout_passes=False),
    )(_sc_body)
    return fn(rec, pred1)


_NROW = _B * _NA
_RSTEP = 8
_NSTEP = _NROW // _RSTEP


def _loss_body(conf_ref, mg_ref, rows_ref, rec_ref, out_ref, acc_ref):
    i = pl.program_id(0)

    @pl.when(i == 0)
    def _():
        acc_ref[...] = jnp.zeros_like(acc_ref)

    def rsum2(v):
        return jnp.sum(v.reshape(2, 8, 128), axis=0)

    conf = conf_ref[...]
    mask = mg_ref[:, 0]
    igng = mg_ref[:, 1]
    ff = (1.0 - mask) * (1.0 - igng)
    bce0 = jnp.maximum(conf, 0.0) + jnp.log1p(jnp.exp(-jnp.abs(conf)))
    acc_ref[1] = acc_ref[1] + jnp.sum(ff, axis=0)
    acc_ref[6] = acc_ref[6] + jnp.sum(ff * bce0, axis=0)

    @pl.when(i == 0)
    def _():
        rows = rows_ref[...]
        rec = rec_ref[...]
        flag = rows[:, 26]
        conf_s = rows[:, 0]
        x = rows[:, 1]
        y = rows[:, 2]
        h = rows[:, 4]
        w = rows[:, 5]
        txg = rec[1]
        tyg = rec[2]
        twg = rec[3]
        thg = rec[4]
        labi = rec[5].astype(jnp.int32)

        bce1 = (jnp.maximum(conf_s, 0.0) - conf_s +
                jnp.log1p(jnp.exp(-jnp.abs(conf_s))))

        cmax = rows[:, 6]
        for c in range(1, _NUM_CLASSES):
            cmax = jnp.maximum(cmax, rows[:, 6 + c])
        esum = jnp.zeros_like(cmax)
        picked = jnp.zeros_like(cmax)
        for c in range(_NUM_CLASSES):
            cls_c = rows[:, 6 + c]
            esum = esum + jnp.exp(cls_c - cmax)
            picked = picked + jnp.where(labi == c, cls_c, 0.0)
        picked = picked - (cmax + jnp.log(esum))

        acc_ref[0] = acc_ref[0] + rsum2(flag)
        acc_ref[2] = acc_ref[2] + rsum2(flag * (x - txg) ** 2)
        acc_ref[3] = acc_ref[3] + rsum2(flag * (y - tyg) ** 2)
        acc_ref[4] = acc_ref[4] + rsum2(flag * (w - twg) ** 2)
        acc_ref[5] = acc_ref[5] + rsum2(flag * (h - thg) ** 2)
        acc_ref[7] = acc_ref[7] + rsum2(flag * bce1)
        acc_ref[8] = acc_ref[8] + rsum2(flag * picked)

    @pl.when(i == _NSTEP - 1)
    def _():
        s = jnp.sum(acc_ref[...], axis=(1, 2))
        nM = s[0]
        nF = s[1]
        total = ((s[2] + s[3] + s[4] + s[5]) / nM +
                 _BAD_CONF_WEIGHT * s[6] / nF + s[7] / nM - s[8] / nM)
        out_ref[...] = jnp.full((1, 1), total, jnp.float32)


def _dense_loss(conf_rows, mg_rows, rows, rec):
    return pl.pallas_call(
        _loss_body,
        grid=(_NSTEP,),
        in_specs=[
            pl.BlockSpec((_RSTEP, 8, 128), lambda i: (i, 0, 0)),
            pl.BlockSpec((_RSTEP, 2, 8, 128), lambda i: (i, 0, 0, 0)),
            pl.BlockSpec((_B, _NROWCH, _TPAD), lambda i: (0, 0, 0)),
            pl.BlockSpec((_NFIELD, _B, _TPAD), lambda i: (0, 0, 0)),
        ],
        out_specs=pl.BlockSpec((1, 1), lambda i: (0, 0)),
        out_shape=jax.ShapeDtypeStruct((1, 1), jnp.float32),
        scratch_shapes=[pltpu.VMEM((9, 8, 128), jnp.float32)],
    )(conf_rows, mg_rows, rows, rec)


def kernel(prediction, target, target_sizes):
    rec = _make_records(target.astype(jnp.float32), target_sizes)
    pred1 = prediction.reshape(_NROW * _CELLS * _NCH)
    mg, conf, rows = _sc_scatter(rec, pred1)
    out = _dense_loss(conf.reshape(_NROW, 8, 128),
                      mg.reshape(_NROW, 2, 8, 128),
                      rows.reshape(_B, _NROWCH, _TPAD), rec)
    return out[0, 0]


# speedup vs baseline: 1.6616x; 1.6616x over previous
import functools

import jax
import jax.numpy as jnp
from jax import lax
from jax.experimental import pallas as pl
from jax.experimental.pallas import tpu as pltpu
from jax.experimental.pallas import tpu_sc as plsc

_NUM_CLASSES = 20
_SCALE = 16.0
_IGNORE_THRESH = 0.5
_BAD_CONF_WEIGHT = 1.25
_ANCHORS = [(25.0, 50.0), (50.0, 100.0), (100.0, 200.0), (200.0, 120.0),
            (320.0, 320.0)]

_B, _NA, _NH, _NW, _MAXT = 16, 5, 32, 32, 50
_CELLS = _NH * _NW
_GRID = _NA * _CELLS
_NFIELD = 16
_TPAD = 128
_NSEC = 7
_COMB = _NSEC * _GRID


def _records_body(tgt_ref, ts_ref, out_ref):
    tgt = tgt_ref[...]
    ts = ts_ref[...]
    inv_s = 1.0 / _SCALE
    gx = tgt[:, :, 0] * inv_s
    gy = tgt[:, :, 1] * inv_s
    gh = tgt[:, :, 3] * inv_s
    gw = tgt[:, :, 4] * inv_s

    tt = lax.broadcasted_iota(jnp.int32, (_B, _MAXT), 1)
    valid = (tt < ts) & (gw != 0.0) & (gh != 0.0)

    gi = jnp.clip(gx.astype(jnp.int32), 0, _NW - 1)
    gj = jnp.clip(gy.astype(jnp.int32), 0, _NH - 1)

    a1 = (gw + 1.0) * (gh + 1.0)
    ious = []
    for aw, ah in _ANCHORS:
        aw, ah = aw / _SCALE, ah / _SCALE
        inter = (jnp.clip(jnp.minimum(gw, aw) + 1.0, 0.0, None) *
                 jnp.clip(jnp.minimum(gh, ah) + 1.0, 0.0, None))
        a2 = (aw + 1.0) * (ah + 1.0)
        ious.append(inter / (a1 + a2 - inter + 1e-16))

    best_iou = ious[0]
    best_n = jnp.zeros((_B, _MAXT), jnp.int32)
    for a in range(1, _NA):
        upd = ious[a] > best_iou
        best_n = jnp.where(upd, a, best_n)
        best_iou = jnp.where(upd, ious[a], best_iou)

    validf = valid.astype(jnp.float32)
    ign = [((iou_a > _IGNORE_THRESH) & valid).astype(jnp.float32)
           for iou_a in ious]

    aw_best = jnp.full((_B, _MAXT), _ANCHORS[0][0] / _SCALE)
    ah_best = jnp.full((_B, _MAXT), _ANCHORS[0][1] / _SCALE)
    for a in range(1, _NA):
        sel = best_n == a
        aw_best = jnp.where(sel, _ANCHORS[a][0] / _SCALE, aw_best)
        ah_best = jnp.where(sel, _ANCHORS[a][1] / _SCALE, ah_best)

    def inv_tanh(y):
        yc = jnp.clip(y, -0.999999, 0.999999)
        inner = 0.5 * jnp.log((1.0 + yc) / (1.0 - yc))
        return jnp.where(y <= -1.0, -2.0, jnp.where(y >= 1.0, 2.0, inner))

    txv = inv_tanh(gx - (gi.astype(jnp.float32) + 0.5))
    tyv = inv_tanh(gy - (gj.astype(jnp.float32) + 0.5))
    twv = jnp.log(gw / aw_best + 1e-16)
    thv = jnp.log(gh / ah_best + 1e-16)

    cidx = lax.broadcasted_iota(
        jnp.int32, (_B, _MAXT, _NUM_CLASSES), 2).astype(jnp.float32)
    label = jnp.sum(tgt[:, :, 13:13 + _NUM_CLASSES] * cidx, axis=2)

    cell = (gj * _NW + gi).astype(jnp.float32)
    key1 = (best_n * (_NSEC * _CELLS)).astype(jnp.float32) + cell

    zeros = jnp.zeros((_B, _MAXT), jnp.float32)
    fields = [validf, txv, tyv, twv, thv, label,
              ign[0], ign[1], ign[2], ign[3], ign[4],
              key1, cell, zeros, zeros, zeros]
    pad = jnp.zeros((_B, _TPAD - _MAXT), jnp.float32)
    for k, f in enumerate(fields):
        out_ref[k] = jnp.concatenate([f, pad], axis=1)


def _make_records(target, target_sizes):
    return pl.pallas_call(
        _records_body,
        out_shape=jax.ShapeDtypeStruct((_NFIELD, _B, _TPAD), jnp.float32),
    )(target, target_sizes.astype(jnp.int32).reshape(_B, 1))


def _sc_scatter_body(rec_hbm, out_hbm, rec_v, comb_v):
    cid = lax.axis_index("c")
    sid = lax.axis_index("s")

    @pl.when(cid == 0)
    def _():
        b = sid
        pltpu.sync_copy(rec_hbm.at[:, b], rec_v)

        def zero_body(i, carry):
            zero = jnp.zeros((16,), jnp.float32)
            for u in range(8):
                comb_v[pl.ds(i * 128 + u * 16, 16)] = zero
            return carry

        lax.fori_loop(0, _COMB // 128, zero_body, 0)

        lane = lax.iota(jnp.int32, 16)

        def tgt_body(t, carry):
            v = plsc.load_gather(rec_v, [lane, jnp.full((16,), t, jnp.int32)])
            valid = v[0]
            key1 = v[11].astype(jnp.int32)
            cell = v[12].astype(jnp.int32)
            validv = jnp.full((16,), valid) > 0.0
            m1 = (lane < 6) & validv
            idx1 = jnp.where(m1, lane * _CELLS + key1, 0)
            plsc.store_scatter(comb_v, [idx1], v, mask=m1)
            m2 = (lane >= 6) & (lane < 11) & (v > 0.0)
            idx2 = jnp.where(
                m2, (lane - 6) * (_NSEC * _CELLS) + 6 * _CELLS + cell, 0)
            plsc.store_scatter(comb_v, [idx2], v, mask=m2)
            return carry

        lax.fori_loop(0, _MAXT, tgt_body, 0)

        pltpu.sync_copy(comb_v, out_hbm.at[pl.ds(b * _COMB, _COMB)])


def _sc_scatter(rec):
    mesh = plsc.VectorSubcoreMesh(core_axis_name="c", subcore_axis_name="s",
                                  num_cores=2, num_subcores=16)
    fn = functools.partial(
        pl.kernel,
        out_type=jax.ShapeDtypeStruct((_B * _COMB,), jnp.float32),
        mesh=mesh,
        scratch_types=[
            pltpu.VMEM((_NFIELD, _TPAD), jnp.float32),
            pltpu.VMEM((_COMB,), jnp.float32),
        ],
        compiler_params=pltpu.CompilerParams(needs_layout_passes=False),
    )(_sc_scatter_body)
    return fn(rec)


_NROW = _B * _NA
_RSTEP = 8
_NSTEP = _NROW // _RSTEP


def _loss_body(pred_ref, grids_ref, out_ref, acc_ref):
    i = pl.program_id(0)

    @pl.when(i == 0)
    def _():
        acc_ref[...] = jnp.zeros_like(acc_ref)

    p = pred_ref[...]
    g = grids_ref[...]
    mask = g[:, 0]
    txg = g[:, 1]
    tyg = g[:, 2]
    twg = g[:, 3]
    thg = g[:, 4]
    labg = g[:, 5]
    igng = g[:, 6]

    conf = p[:, 0]
    x = p[:, 1]
    y = p[:, 2]
    h = p[:, 4]
    w = p[:, 5]

    ff = (1.0 - mask) * (1.0 - igng)

    bce = (jnp.maximum(conf, 0.0) - conf * mask +
           jnp.log1p(jnp.exp(-jnp.abs(conf))))

    cmax = p[:, 6]
    for c in range(1, _NUM_CLASSES):
        cmax = jnp.maximum(cmax, p[:, 6 + c])
    esum = jnp.zeros_like(cmax)
    picked = jnp.zeros_like(cmax)
    labi = labg.astype(jnp.int32)
    for c in range(_NUM_CLASSES):
        cls_c = p[:, 6 + c]
        esum = esum + jnp.exp(cls_c - cmax)
        picked = picked + jnp.where(labi == c, cls_c, 0.0)
    lse = cmax + jnp.log(esum)
    picked = picked - lse

    def rsum(v):
        return jnp.sum(v, axis=0)

    acc = acc_ref[...]
    terms = [mask,
             ff,
             mask * (x - txg) ** 2,
             mask * (y - tyg) ** 2,
             mask * (w - twg) ** 2,
             mask * (h - thg) ** 2,
             ff * bce,
             mask * bce,
             mask * picked]
    acc_ref[...] = acc + jnp.stack([rsum(t) for t in terms], axis=0)

    @pl.when(i == _NSTEP - 1)
    def _():
        s = jnp.sum(acc_ref[...], axis=(1, 2))
        nM = s[0]
        nF = s[1]
        total = ((s[2] + s[3] + s[4] + s[5]) / nM +
                 _BAD_CONF_WEIGHT * s[6] / nF + s[7] / nM - s[8] / nM)
        out_ref[...] = jnp.full((1, 1), total, jnp.float32)


def _dense_loss(pred_rows, grid_rows):
    return pl.pallas_call(
        _loss_body,
        grid=(_NSTEP,),
        in_specs=[
            pl.BlockSpec((_RSTEP, 6 + _NUM_CLASSES, 8, 128),
                         lambda i: (i, 0, 0, 0)),
            pl.BlockSpec((_RSTEP, _NSEC, 8, 128), lambda i: (i, 0, 0, 0)),
        ],
        out_specs=pl.BlockSpec((1, 1), lambda i: (0, 0)),
        out_shape=jax.ShapeDtypeStruct((1, 1), jnp.float32),
        scratch_shapes=[pltpu.VMEM((9, 8, 128), jnp.float32)],
    )(pred_rows, grid_rows)


def kernel(prediction, target, target_sizes):
    rec = _make_records(target.astype(jnp.float32), target_sizes)
    grids = _sc_scatter(rec)
    pred_rows = prediction.transpose(0, 1, 4, 2, 3).reshape(
        _NROW, 6 + _NUM_CLASSES, 8, _CELLS // 8)
    grid_rows = grids.reshape(_NROW, _NSEC, 8, _NH * _NW // 8)
    out = _dense_loss(pred_rows, grid_rows)
    return out[0, 0]


# speedup vs baseline: 1.7140x; 1.0316x over previous
import functools

import jax
import jax.numpy as jnp
from jax import lax
from jax.experimental import pallas as pl
from jax.experimental.pallas import tpu as pltpu
from jax.experimental.pallas import tpu_sc as plsc

_NUM_CLASSES = 20
_SCALE = 16.0
_IGNORE_THRESH = 0.5
_BAD_CONF_WEIGHT = 1.25
_ANCHORS = [(25.0, 50.0), (50.0, 100.0), (100.0, 200.0), (200.0, 120.0),
            (320.0, 320.0)]

_B, _NA, _NH, _NW, _MAXT = 16, 5, 32, 32, 50
_CELLS = _NH * _NW
_GRID = _NA * _CELLS
_NFIELD = 16
_TPAD = 128
_NSEC = 7
_COMB = _NSEC * _GRID


def _records_body(tgt_ref, ts_ref, out_ref):
    tgt = tgt_ref[...]
    ts = ts_ref[...]
    inv_s = 1.0 / _SCALE
    gx = tgt[:, :, 0] * inv_s
    gy = tgt[:, :, 1] * inv_s
    gh = tgt[:, :, 3] * inv_s
    gw = tgt[:, :, 4] * inv_s

    tt = lax.broadcasted_iota(jnp.int32, (_B, _MAXT), 1)
    valid = (tt < ts) & (gw != 0.0) & (gh != 0.0)

    gi = jnp.clip(gx.astype(jnp.int32), 0, _NW - 1)
    gj = jnp.clip(gy.astype(jnp.int32), 0, _NH - 1)

    a1 = (gw + 1.0) * (gh + 1.0)
    ious = []
    for aw, ah in _ANCHORS:
        aw, ah = aw / _SCALE, ah / _SCALE
        inter = (jnp.clip(jnp.minimum(gw, aw) + 1.0, 0.0, None) *
                 jnp.clip(jnp.minimum(gh, ah) + 1.0, 0.0, None))
        a2 = (aw + 1.0) * (ah + 1.0)
        ious.append(inter / (a1 + a2 - inter + 1e-16))

    best_iou = ious[0]
    best_n = jnp.zeros((_B, _MAXT), jnp.int32)
    for a in range(1, _NA):
        upd = ious[a] > best_iou
        best_n = jnp.where(upd, a, best_n)
        best_iou = jnp.where(upd, ious[a], best_iou)

    validf = valid.astype(jnp.float32)
    ign = [((iou_a > _IGNORE_THRESH) & valid).astype(jnp.float32)
           for iou_a in ious]

    aw_best = jnp.full((_B, _MAXT), _ANCHORS[0][0] / _SCALE)
    ah_best = jnp.full((_B, _MAXT), _ANCHORS[0][1] / _SCALE)
    for a in range(1, _NA):
        sel = best_n == a
        aw_best = jnp.where(sel, _ANCHORS[a][0] / _SCALE, aw_best)
        ah_best = jnp.where(sel, _ANCHORS[a][1] / _SCALE, ah_best)

    def inv_tanh(y):
        yc = jnp.clip(y, -0.999999, 0.999999)
        inner = 0.5 * jnp.log((1.0 + yc) / (1.0 - yc))
        return jnp.where(y <= -1.0, -2.0, jnp.where(y >= 1.0, 2.0, inner))

    txv = inv_tanh(gx - (gi.astype(jnp.float32) + 0.5))
    tyv = inv_tanh(gy - (gj.astype(jnp.float32) + 0.5))
    twv = jnp.log(gw / aw_best + 1e-16)
    thv = jnp.log(gh / ah_best + 1e-16)

    cidx = lax.broadcasted_iota(
        jnp.int32, (_B, _MAXT, _NUM_CLASSES), 2).astype(jnp.float32)
    label = jnp.sum(tgt[:, :, 13:13 + _NUM_CLASSES] * cidx, axis=2)

    cell = (gj * _NW + gi).astype(jnp.float32)
    key1 = (best_n * (_NSEC * _CELLS)).astype(jnp.float32) + cell

    zeros = jnp.zeros((_B, _MAXT), jnp.float32)
    fields = [validf, txv, tyv, twv, thv, label,
              ign[0], ign[1], ign[2], ign[3], ign[4],
              key1, cell, zeros, zeros, zeros]
    pad = jnp.zeros((_B, _TPAD - _MAXT), jnp.float32)
    for k, f in enumerate(fields):
        out_ref[k] = jnp.concatenate([f, pad], axis=1)


def _make_records(target, target_sizes):
    return pl.pallas_call(
        _records_body,
        out_shape=jax.ShapeDtypeStruct((_NFIELD, _B, _TPAD), jnp.float32),
    )(target, target_sizes.astype(jnp.int32).reshape(_B, 1))


def _sc_scatter_body(rec_hbm, out_hbm, rec_v, comb_v):
    cid = lax.axis_index("c")
    sid = lax.axis_index("s")

    @pl.when(cid == 0)
    def _():
        b = sid
        pltpu.sync_copy(rec_hbm.at[:, b], rec_v)

        def zero_body(i, carry):
            zero = jnp.zeros((16,), jnp.float32)
            for u in range(8):
                comb_v[pl.ds(i * 128 + u * 16, 16)] = zero
            return carry

        lax.fori_loop(0, _COMB // 128, zero_body, 0)

        lane = lax.iota(jnp.int32, 16)

        def tgt_body(t, carry):
            v = plsc.load_gather(rec_v, [lane, jnp.full((16,), t, jnp.int32)])
            valid = v[0]
            key1 = v[11].astype(jnp.int32)
            cell = v[12].astype(jnp.int32)
            validv = jnp.full((16,), valid) > 0.0
            m1 = (lane < 6) & validv
            idx1 = jnp.where(m1, lane * _CELLS + key1, 0)
            plsc.store_scatter(comb_v, [idx1], v, mask=m1)
            m2 = (lane >= 6) & (lane < 11) & (v > 0.0)
            idx2 = jnp.where(
                m2, (lane - 6) * (_NSEC * _CELLS) + 6 * _CELLS + cell, 0)
            plsc.store_scatter(comb_v, [idx2], v, mask=m2)
            return carry

        lax.fori_loop(0, _MAXT, tgt_body, 0)

        pltpu.sync_copy(comb_v, out_hbm.at[pl.ds(b * _COMB, _COMB)])


def _sc_scatter(rec):
    mesh = plsc.VectorSubcoreMesh(core_axis_name="c", subcore_axis_name="s",
                                  num_cores=2, num_subcores=16)
    fn = functools.partial(
        pl.kernel,
        out_type=jax.ShapeDtypeStruct((_B * _COMB,), jnp.float32),
        mesh=mesh,
        scratch_types=[
            pltpu.VMEM((_NFIELD, _TPAD), jnp.float32),
            pltpu.VMEM((_COMB,), jnp.float32),
        ],
        compiler_params=pltpu.CompilerParams(needs_layout_passes=False),
    )(_sc_scatter_body)
    return fn(rec)


_NROW = _B * _NA
_RSTEP = 16
_NSTEP = _NROW // _RSTEP


def _loss_body(pred_ref, grids_ref, out_ref, acc_ref):
    i = pl.program_id(0)

    @pl.when(i == 0)
    def _():
        acc_ref[...] = jnp.zeros_like(acc_ref)

    p = pred_ref[...]
    g = grids_ref[...]
    mask = g[:, 0]
    txg = g[:, 1]
    tyg = g[:, 2]
    twg = g[:, 3]
    thg = g[:, 4]
    labg = g[:, 5]
    igng = g[:, 6]

    conf = p[:, 0]
    x = p[:, 1]
    y = p[:, 2]
    h = p[:, 4]
    w = p[:, 5]

    ff = (1.0 - mask) * (1.0 - igng)

    bce = (jnp.maximum(conf, 0.0) - conf * mask +
           jnp.log1p(jnp.exp(-jnp.abs(conf))))

    cmax = p[:, 6]
    for c in range(1, _NUM_CLASSES):
        cmax = jnp.maximum(cmax, p[:, 6 + c])
    esum = jnp.zeros_like(cmax)
    picked = jnp.zeros_like(cmax)
    labi = labg.astype(jnp.int32)
    for c in range(_NUM_CLASSES):
        cls_c = p[:, 6 + c]
        esum = esum + jnp.exp(cls_c - cmax)
        picked = picked + jnp.where(labi == c, cls_c, 0.0)
    lse = cmax + jnp.log(esum)
    picked = picked - lse

    def rsum(v):
        return jnp.sum(v, axis=0)

    acc = acc_ref[...]
    terms = [mask,
             ff,
             mask * (x - txg) ** 2,
             mask * (y - tyg) ** 2,
             mask * (w - twg) ** 2,
             mask * (h - thg) ** 2,
             ff * bce,
             mask * bce,
             mask * picked]
    acc_ref[...] = acc + jnp.stack([rsum(t) for t in terms], axis=0)

    @pl.when(i == _NSTEP - 1)
    def _():
        s = jnp.sum(acc_ref[...], axis=(1, 2))
        nM = s[0]
        nF = s[1]
        total = ((s[2] + s[3] + s[4] + s[5]) / nM +
                 _BAD_CONF_WEIGHT * s[6] / nF + s[7] / nM - s[8] / nM)
        out_ref[...] = jnp.full((1, 1), total, jnp.float32)


def _dense_loss(pred_rows, grid_rows):
    return pl.pallas_call(
        _loss_body,
        grid=(_NSTEP,),
        in_specs=[
            pl.BlockSpec((_RSTEP, 6 + _NUM_CLASSES, 8, 128),
                         lambda i: (i, 0, 0, 0)),
            pl.BlockSpec((_RSTEP, _NSEC, 8, 128), lambda i: (i, 0, 0, 0)),
        ],
        out_specs=pl.BlockSpec((1, 1), lambda i: (0, 0)),
        out_shape=jax.ShapeDtypeStruct((1, 1), jnp.float32),
        scratch_shapes=[pltpu.VMEM((9, 8, 128), jnp.float32)],
    )(pred_rows, grid_rows)


def kernel(prediction, target, target_sizes):
    rec = _make_records(target.astype(jnp.float32), target_sizes)
    grids = _sc_scatter(rec)
    pred_rows = prediction.transpose(0, 1, 4, 2, 3).reshape(
        _NROW, 6 + _NUM_CLASSES, 8, _CELLS // 8)
    grid_rows = grids.reshape(_NROW, _NSEC, 8, _NH * _NW // 8)
    out = _dense_loss(pred_rows, grid_rows)
    return out[0, 0]
